# baseline (speedup 1.0000x reference)
import functools

import jax
import jax.numpy as jnp
from jax import lax
from jax.experimental import pallas as pl
from jax.experimental.pallas import tpu as pltpu
from jax.experimental.pallas import tpu_sc as plsc

N = 10000
E = 160000
D = 256
H = 2
C = 128
NEG = 0.2
L = 16
NSUB = 16
CHUNK = 64
NGRP = CHUNK // L
NCHG = E // CHUNK
JPAD = (NCHG + NSUB - 1) // NSUB
NPAD = JPAD * CHUNK
WFULL = N // CHUNK
WTAIL = N - WFULL * CHUNK


def _mm_body(x_ref, w_ref, o_ref):
    o_ref[0] = jnp.dot(x_ref[...], w_ref[...],
                       preferred_element_type=jnp.float32)


def _mm_body2(x_ref, w_ref, o_ref):
    o_ref[...] = jnp.dot(x_ref[...], w_ref[...],
                         preferred_element_type=jnp.float32)


def _project_w(x, w, bn):
    M, K = x.shape
    G = w.shape[1] // 128
    nb = M // bn
    return pl.pallas_call(
        _mm_body2,
        grid=(G, nb),
        in_specs=[
            pl.BlockSpec((bn, K), lambda g, i: (i, 0)),
            pl.BlockSpec((K, 128), lambda g, i: (0, g)),
        ],
        out_specs=pl.BlockSpec((bn, 128), lambda g, i: (g * (M // bn) + i, 0)),
        out_shape=jax.ShapeDtypeStruct((G * M, 128), jnp.float32),
    )(x, w)


def _project(x, w, bn):
    M, K = x.shape
    G = w.shape[1] // 128
    return pl.pallas_call(
        _mm_body,
        grid=(G, M // bn),
        in_specs=[
            pl.BlockSpec((bn, K), lambda g, i: (i, 0)),
            pl.BlockSpec((K, 128), lambda g, i: (0, g)),
        ],
        out_specs=pl.BlockSpec((1, bn, 128), lambda g, i: (g, i, 0)),
        out_shape=jax.ShapeDtypeStruct((G, M, 128), jnp.float32),
    )(x, w)


def _sc_body(e0, e1, paf, pbf, eef, attb, biasb,
             outs, outt,
             xlb0, xlb1, xrb0, xrb1, eeb,
             eib0, eib1, eib2, eib3,
             exb0, exb1, attv, biasv,
             out_sh, denf_sh,
             semxl0, semxl1, semxr0, semxr1, semee,
             semso0, semso1, semsd0, semsd1,
             semei0, semei1, semei2, semei3):
    c = lax.axis_index("c")
    tid = lax.axis_index("s")
    iota = lax.iota(jnp.int32, L)
    zv = jnp.zeros((L,), jnp.float32)
    lane0 = iota == 0
    xlb = (xlb0, xlb1)
    xrb = (xrb0, xrb1)
    eib = (eib0, eib1, eib2, eib3)
    semei = (semei0, semei1, semei2, semei3)
    semxl = (semxl0, semxl1)
    semxr = (semxr0, semxr1)
    semso = (semso0, semso1)
    semsd = (semsd0, semsd1)
    exb = (exb0, exb1)

    for d, (xl, xr, esrc, edst, outref) in enumerate((
            (paf, pbf, e0, e1, outs),
            (pbf, paf, e1, e0, outt))):
        q = d * 2 + c
        pltpu.sync_copy(attb.at[pl.ds(q * C, C)], attv)
        pltpu.sync_copy(biasb.at[pl.ds(q * C, C)], biasv)
        hoff = q * N

        def _zrow(r, _):
            for jz in range(C // L):
                xlb0[r, pl.ds(jz * L, L)] = zv
            return 0
        lax.fori_loop(0, CHUNK, _zrow, 0)
        for g in range(NGRP):
            exb0[pl.ds(g * L, L)] = zv

        def _zout(jz, _):
            k = tid + jz * NSUB

            @pl.when(k < WFULL)
            def _():
                pltpu.sync_copy(xlb0, out_sh.at[pl.ds(k * CHUNK, CHUNK)])

            @pl.when(k == WFULL)
            def _():
                pltpu.sync_copy(xlb0.at[pl.ds(0, WTAIL)],
                                out_sh.at[pl.ds(WFULL * CHUNK, WTAIL)])

            @pl.when(k < JPAD)
            def _():
                pltpu.sync_copy(exb0, denf_sh.at[pl.ds(k * CHUNK, CHUNK)])
            return 0
        lax.fori_loop(0, (JPAD + NSUB - 1) // NSUB, _zout, 0)

        plsc.subcore_barrier()

        def _ifetch(j, qs):
            k = tid + j * NSUB
            base = jnp.minimum(k, NCHG - 1) * CHUNK
            pltpu.async_copy(esrc.at[pl.ds(base, CHUNK)], eib[qs].at[0],
                             semei[qs])
            pltpu.async_copy(edst.at[pl.ds(base, CHUNK)], eib[qs].at[1],
                             semei[qs])

        def _gissue(j, qs, b):
            k = tid + j * NSUB
            base = jnp.minimum(k, NCHG - 1) * CHUNK
            pltpu.make_async_copy(esrc.at[pl.ds(base, CHUNK)],
                                  eib[qs].at[0], semei[qs]).wait()
            pltpu.make_async_copy(edst.at[pl.ds(base, CHUNK)],
                                  eib[qs].at[1], semei[qs]).wait()
            for g in range(NGRP):
                sl = pl.ds(g * L, L)
                eib[qs][0, sl] = eib[qs][0, sl] + hoff
                eib[qs][1, sl] = eib[qs][1, sl] + hoff
            pltpu.async_copy(xl.at[eib[qs].at[0]], xlb[b], semxl[b])
            pltpu.async_copy(xr.at[eib[qs].at[1]], xrb[b], semxr[b])

        def _compute(j, qs, b):
            valid = (tid + j * NSUB) < NCHG
            vs = jnp.full((L,), jnp.where(valid, 1.0, 0.0), jnp.float32)
            mxl = xlb[b]
            mxr = xrb[b]
            mee = eeb

            @plsc.parallel_loop(0, CHUNK, unroll=4)
            def _alpha(e):
                acc = zv
                for jj in range(C // L):
                    sl = pl.ds(jj * L, L)
                    m = mxl[e, sl] + mxr[e, sl] + mee[e, sl]
                    m = jnp.maximum(m, NEG * m)
                    acc = acc + m * attv[sl]
                ex = jnp.exp(jnp.full((L,), jnp.sum(acc), jnp.float32)) * vs
                plsc.store_scatter(exb[b], [jnp.full((L,), e, jnp.int32)],
                                   ex, mask=lane0)

            @pl.when(j + 1 < JPAD)
            def _():
                k1 = tid + (j + 1) * NSUB
                base1 = jnp.minimum(k1, NCHG - 1) * CHUNK
                pltpu.async_copy(eef.at[pl.ds(q * E + base1, CHUNK)], eeb,
                                 semee)

            @plsc.parallel_loop(0, CHUNK, unroll=4)
            def _scale(e):
                exv = plsc.load_gather(exb[b],
                                       [jnp.full((L,), e, jnp.int32)])
                for jj in range(C // L):
                    sl = pl.ds(jj * L, L)
                    mxl[e, sl] = mxl[e, sl] * exv

            for g in range(NGRP):
                sl = pl.ds(g * L, L)
                eib[qs][1, sl] = eib[qs][1, sl] - hoff
            pltpu.async_copy(exb[b], denf_sh.at[eib[qs].at[1]], semsd[b],
                             add=True)
            pltpu.async_copy(mxl, out_sh.at[eib[qs].at[1]], semso[b],
                             add=True)

        _ifetch(0, 0)
        _ifetch(1, 1)
        _ifetch(2, 2)
        _gissue(0, 0, 0)
        base00 = jnp.minimum(tid, NCHG - 1) * CHUNK
        pltpu.async_copy(eef.at[pl.ds(q * E + base00, CHUNK)], eeb, semee)

        def _quad(jo, _):
            for b4 in range(4):
                j = jo * 4 + b4
                b = b4 % 2

                @pl.when(j < JPAD)
                def _():
                    k = tid + j * NSUB
                    base = jnp.minimum(k, NCHG - 1) * CHUNK
                    pltpu.make_async_copy(xl.at[eib[b4].at[0]], xlb[b],
                                          semxl[b]).wait()
                    pltpu.make_async_copy(xr.at[eib[b4].at[1]], xrb[b],
                                          semxr[b]).wait()
                    pltpu.make_async_copy(
                        eef.at[pl.ds(q * E + base, CHUNK)], eeb,
                        semee).wait()

                    @pl.when(j + 3 < JPAD)
                    def _():
                        _ifetch(j + 3, (b4 + 3) % 4)

                    @pl.when(j + 1 < JPAD)
                    def _():
                        @pl.when(j >= 1)
                        def _():
                            pltpu.make_async_copy(
                                exb[1 - b],
                                denf_sh.at[eib[(b4 + 3) % 4].at[1]],
                                semsd[1 - b]).wait()
                            pltpu.make_async_copy(
                                xlb[1 - b],
                                out_sh.at[eib[(b4 + 3) % 4].at[1]],
                                semso[1 - b]).wait()
                        _gissue(j + 1, (b4 + 1) % 4, 1 - b)
                    _compute(j, b4, b)
            return 0
        lax.fori_loop(0, (JPAD + 3) // 4, _quad, 0)
        for m in (JPAD - 2, JPAD - 1):
            bm = m % 2
            qm = m % 4
            pltpu.make_async_copy(exb[bm], denf_sh.at[eib[qm].at[1]],
                                  semsd[bm]).wait()
            pltpu.make_async_copy(xlb[bm], out_sh.at[eib[qm].at[1]],
                                  semso[bm]).wait()

        plsc.subcore_barrier()

        def _wb(rows, rowbase):
            pltpu.sync_copy(out_sh.at[pl.ds(rowbase, rows)],
                            xrb0.at[pl.ds(0, rows)])
            pltpu.sync_copy(denf_sh.at[pl.ds(rowbase, rows)],
                            exb0.at[pl.ds(0, rows)])

            @plsc.parallel_loop(0, rows, unroll=4)
            def _nrm(r):
                dv = plsc.load_gather(exb0,
                                      [jnp.full((L,), r, jnp.int32)])
                rcv = 1.0 / (dv + 1e-16)
                for jj in range(C // L):
                    sl = pl.ds(jj * L, L)
                    v = xrb0[r, sl] * rcv + biasv[sl]
                    xrb0[r, sl] = jnp.maximum(v, 0.0)
            pltpu.sync_copy(
                xrb0.at[pl.ds(0, rows)],
                outref.at[pl.ds(rowbase, rows), pl.ds(c * C, C)])

        def _wchunk(jw, _):
            k = tid + jw * NSUB

            @pl.when(k < WFULL)
            def _():
                _wb(CHUNK, k * CHUNK)

            @pl.when(k == WFULL)
            def _():
                _wb(WTAIL, WFULL * CHUNK)
            return 0
        lax.fori_loop(0, (WFULL + NSUB) // NSUB, _wchunk, 0)
        plsc.subcore_barrier()


_sc_call = pl.kernel(
    _sc_body,
    out_type=(
        jax.ShapeDtypeStruct((N, H * C), jnp.float32),
        jax.ShapeDtypeStruct((N, H * C), jnp.float32),
    ),
    mesh=plsc.VectorSubcoreMesh(core_axis_name="c", subcore_axis_name="s"),
    compiler_params=pltpu.CompilerParams(needs_layout_passes=False),
    scratch_types=[
        pltpu.VMEM((CHUNK, C), jnp.float32),
        pltpu.VMEM((CHUNK, C), jnp.float32),
        pltpu.VMEM((CHUNK, C), jnp.float32),
        pltpu.VMEM((CHUNK, C), jnp.float32),
        pltpu.VMEM((CHUNK, C), jnp.float32),
        pltpu.VMEM((2, CHUNK), jnp.int32),
        pltpu.VMEM((2, CHUNK), jnp.int32),
        pltpu.VMEM((2, CHUNK), jnp.int32),
        pltpu.VMEM((2, CHUNK), jnp.int32),
        pltpu.VMEM((CHUNK,), jnp.float32),
        pltpu.VMEM((CHUNK,), jnp.float32),
        pltpu.VMEM((C,), jnp.float32),
        pltpu.VMEM((C,), jnp.float32),
        pltpu.VMEM_SHARED((N, C), jnp.float32),
        pltpu.VMEM_SHARED((NPAD,), jnp.float32),
        pltpu.SemaphoreType.DMA,
        pltpu.SemaphoreType.DMA,
        pltpu.SemaphoreType.DMA,
        pltpu.SemaphoreType.DMA,
        pltpu.SemaphoreType.DMA,
        pltpu.SemaphoreType.DMA,
        pltpu.SemaphoreType.DMA,
        pltpu.SemaphoreType.DMA,
        pltpu.SemaphoreType.DMA,
        pltpu.SemaphoreType.DMA,
        pltpu.SemaphoreType.DMA,
        pltpu.SemaphoreType.DMA,
        pltpu.SemaphoreType.DMA,
    ],
)


@jax.jit
def kernel(s, t, edges, edge_weight,
           sWl, sWr, sWe, satt, sbias,
           tWl, tWr, tWe, tatt, tbias):
    pa = _project(s, jnp.concatenate([sWl, tWr], axis=1), 1000)
    pb = _project(t, jnp.concatenate([sWr, tWl], axis=1), 1000)
    paf = pa.reshape(4 * N, C)
    pbf = pb.reshape(4 * N, C)
    eef = _project_w(edge_weight, jnp.concatenate([sWe, tWe], axis=1),
                     2000)

    att4 = jnp.concatenate([satt, tatt], axis=0)
    attb = att4.reshape(4 * C)
    bias4 = jnp.concatenate(
        [sbias.reshape(H, C), tbias.reshape(H, C)], axis=0)
    biasb = bias4.reshape(4 * C)

    outs, outt = _sc_call(edges[0], edges[1], paf, pbf, eef, attb, biasb)
    return (outs, outt, edges, edge_weight)


# speedup vs baseline: 1.1219x; 1.1219x over previous
import functools

import jax
import jax.numpy as jnp
from jax import lax
from jax.experimental import pallas as pl
from jax.experimental.pallas import tpu as pltpu
from jax.experimental.pallas import tpu_sc as plsc

N = 10000
E = 160000
D = 256
H = 2
C = 128
NEG = 0.2
L = 16
NSUB = 16
CHUNK = 64
NGRP = CHUNK // L
NCHG = E // CHUNK
JPAD = (NCHG + NSUB - 1) // NSUB
NPAD = JPAD * CHUNK
WFULL = N // CHUNK
WTAIL = N - WFULL * CHUNK


def _mm_body(x_ref, w_ref, o_ref):
    o_ref[0] = jnp.dot(x_ref[...], w_ref[...],
                       preferred_element_type=jnp.float32)


def _mm_body2(x_ref, w_ref, o_ref):
    o_ref[...] = jnp.dot(x_ref[...], w_ref[...],
                         preferred_element_type=jnp.float32)


def _project_w(x, w, bn):
    M, K = x.shape
    G = w.shape[1] // 128
    nb = M // bn
    return pl.pallas_call(
        _mm_body2,
        grid=(G, nb),
        in_specs=[
            pl.BlockSpec((bn, K), lambda g, i: (i, 0)),
            pl.BlockSpec((K, 128), lambda g, i: (0, g)),
        ],
        out_specs=pl.BlockSpec((bn, 128), lambda g, i: (g * (M // bn) + i, 0)),
        out_shape=jax.ShapeDtypeStruct((G * M, 128), jnp.float32),
    )(x, w)


def _project(x, w, bn):
    M, K = x.shape
    G = w.shape[1] // 128
    return pl.pallas_call(
        _mm_body,
        grid=(G, M // bn),
        in_specs=[
            pl.BlockSpec((bn, K), lambda g, i: (i, 0)),
            pl.BlockSpec((K, 128), lambda g, i: (0, g)),
        ],
        out_specs=pl.BlockSpec((1, bn, 128), lambda g, i: (g, i, 0)),
        out_shape=jax.ShapeDtypeStruct((G, M, 128), jnp.float32),
    )(x, w)


def _sc_body(e0, e1, paf, pbf, eef, attb, biasb,
             outs, outt,
             xlb0, xlb1, xrb0, xrb1, eeb,
             eib0, eib1, eib2, eib3,
             exb0, exb1, attv, biasv,
             out_sh, denf_sh,
             semxl0, semxl1, semxr0, semxr1, semee,
             semso0, semso1, semsd0, semsd1,
             semei0, semei1, semei2, semei3):
    c = lax.axis_index("c")
    tid = lax.axis_index("s")
    iota = lax.iota(jnp.int32, L)
    zv = jnp.zeros((L,), jnp.float32)
    lane0 = iota == 0
    xlb = (xlb0, xlb1)
    xrb = (xrb0, xrb1)
    eib = (eib0, eib1, eib2, eib3)
    semei = (semei0, semei1, semei2, semei3)
    semxl = (semxl0, semxl1)
    semxr = (semxr0, semxr1)
    semso = (semso0, semso1)
    semsd = (semsd0, semsd1)
    exb = (exb0, exb1)

    for d, (xl, xr, esrc, edst, outref) in enumerate((
            (paf, pbf, e0, e1, outs),
            (pbf, paf, e1, e0, outt))):
        q = d * 2 + c
        pltpu.sync_copy(attb.at[pl.ds(q * C, C)], attv)
        pltpu.sync_copy(biasb.at[pl.ds(q * C, C)], biasv)
        hoff = q * N

        def _zrow(r, _):
            for jz in range(C // L):
                xlb0[r, pl.ds(jz * L, L)] = zv
            return 0
        lax.fori_loop(0, CHUNK, _zrow, 0)
        for g in range(NGRP):
            exb0[pl.ds(g * L, L)] = zv

        def _zout(jz, _):
            k = tid + jz * NSUB

            @pl.when(k < WFULL)
            def _():
                pltpu.sync_copy(xlb0, out_sh.at[pl.ds(k * CHUNK, CHUNK)])

            @pl.when(k == WFULL)
            def _():
                pltpu.sync_copy(xlb0.at[pl.ds(0, WTAIL)],
                                out_sh.at[pl.ds(WFULL * CHUNK, WTAIL)])

            @pl.when(k < JPAD)
            def _():
                pltpu.sync_copy(exb0, denf_sh.at[pl.ds(k * CHUNK, CHUNK)])
            return 0
        lax.fori_loop(0, (JPAD + NSUB - 1) // NSUB, _zout, 0)

        plsc.subcore_barrier()

        def _ifetch(j, qs):
            k = tid + j * NSUB
            base = jnp.minimum(k, NCHG - 1) * CHUNK
            pltpu.async_copy(esrc.at[pl.ds(base, CHUNK)], eib[qs].at[0],
                             semei[qs])
            pltpu.async_copy(edst.at[pl.ds(base, CHUNK)], eib[qs].at[1],
                             semei[qs])

        def _gissue(j, qs, b):
            k = tid + j * NSUB
            base = jnp.minimum(k, NCHG - 1) * CHUNK
            pltpu.make_async_copy(esrc.at[pl.ds(base, CHUNK)],
                                  eib[qs].at[0], semei[qs]).wait()
            pltpu.make_async_copy(edst.at[pl.ds(base, CHUNK)],
                                  eib[qs].at[1], semei[qs]).wait()
            for g in range(NGRP):
                sl = pl.ds(g * L, L)
                eib[qs][0, sl] = eib[qs][0, sl] + hoff
                eib[qs][1, sl] = eib[qs][1, sl] + hoff
            pltpu.async_copy(xl.at[eib[qs].at[0]], xlb[b], semxl[b])
            pltpu.async_copy(xr.at[eib[qs].at[1]], xrb[b], semxr[b])

        def _compute(j, qs, b):
            valid = (tid + j * NSUB) < NCHG
            vs = jnp.full((L,), jnp.where(valid, 1.0, 0.0), jnp.float32)
            mxl = xlb[b]
            mxr = xrb[b]
            mee = eeb

            @plsc.parallel_loop(0, CHUNK, unroll=4)
            def _alpha(e):
                acc = zv
                for jj in range(C // L):
                    sl = pl.ds(jj * L, L)
                    m = mxl[e, sl] + mxr[e, sl] + mee[e, sl]
                    m = jnp.maximum(m, NEG * m)
                    acc = acc + m * attv[sl]
                ex = jnp.exp(jnp.full((L,), jnp.sum(acc), jnp.float32)) * vs
                plsc.store_scatter(exb[b], [jnp.full((L,), e, jnp.int32)],
                                   ex, mask=lane0)

            @pl.when(j + 1 < JPAD)
            def _():
                k1 = tid + (j + 1) * NSUB
                base1 = jnp.minimum(k1, NCHG - 1) * CHUNK
                pltpu.async_copy(eef.at[pl.ds(q * E + base1, CHUNK)], eeb,
                                 semee)

            @plsc.parallel_loop(0, CHUNK, unroll=4)
            def _scale(e):
                exv = plsc.load_gather(exb[b],
                                       [jnp.full((L,), e, jnp.int32)])
                for jj in range(C // L):
                    sl = pl.ds(jj * L, L)
                    mxl[e, sl] = mxl[e, sl] * exv

            for g in range(NGRP):
                sl = pl.ds(g * L, L)
                eib[qs][1, sl] = eib[qs][1, sl] - hoff
            pltpu.async_copy(exb[b], denf_sh.at[eib[qs].at[1]], semsd[b],
                             add=True)
            pltpu.async_copy(mxl, out_sh.at[eib[qs].at[1]], semso[b],
                             add=True)

        _ifetch(0, 0)
        _ifetch(1, 1)
        _ifetch(2, 2)
        _gissue(0, 0, 0)
        base00 = jnp.minimum(tid, NCHG - 1) * CHUNK
        pltpu.async_copy(eef.at[pl.ds(q * E + base00, CHUNK)], eeb, semee)

        def _quad(jo, _):
            for b4 in range(4):
                j = jo * 4 + b4
                b = b4 % 2

                @pl.when(j < JPAD)
                def _():
                    k = tid + j * NSUB
                    base = jnp.minimum(k, NCHG - 1) * CHUNK
                    pltpu.make_async_copy(xl.at[eib[b4].at[0]], xlb[b],
                                          semxl[b]).wait()
                    pltpu.make_async_copy(xr.at[eib[b4].at[1]], xrb[b],
                                          semxr[b]).wait()
                    pltpu.make_async_copy(
                        eef.at[pl.ds(q * E + base, CHUNK)], eeb,
                        semee).wait()

                    @pl.when(j + 3 < JPAD)
                    def _():
                        _ifetch(j + 3, (b4 + 3) % 4)

                    @pl.when(j + 1 < JPAD)
                    def _():
                        @pl.when(j >= 1)
                        def _():
                            pltpu.make_async_copy(
                                exb[1 - b],
                                denf_sh.at[eib[(b4 + 3) % 4].at[1]],
                                semsd[1 - b]).wait()
                            pltpu.make_async_copy(
                                xlb[1 - b],
                                out_sh.at[eib[(b4 + 3) % 4].at[1]],
                                semso[1 - b]).wait()
                        _gissue(j + 1, (b4 + 1) % 4, 1 - b)
                    _compute(j, b4, b)
            return 0
        lax.fori_loop(0, (JPAD + 3) // 4, _quad, 0)
        for m in (JPAD - 2, JPAD - 1):
            bm = m % 2
            qm = m % 4
            pltpu.make_async_copy(exb[bm], denf_sh.at[eib[qm].at[1]],
                                  semsd[bm]).wait()
            pltpu.make_async_copy(xlb[bm], out_sh.at[eib[qm].at[1]],
                                  semso[bm]).wait()

        plsc.subcore_barrier()

        def _wb(rows, rowbase):
            pltpu.sync_copy(out_sh.at[pl.ds(rowbase, rows)],
                            xrb0.at[pl.ds(0, rows)])
            pltpu.sync_copy(denf_sh.at[pl.ds(rowbase, rows)],
                            exb0.at[pl.ds(0, rows)])

            @plsc.parallel_loop(0, rows, unroll=4)
            def _nrm(r):
                dv = plsc.load_gather(exb0,
                                      [jnp.full((L,), r, jnp.int32)])
                rcv = 1.0 / (dv + 1e-16)
                for jj in range(C // L):
                    sl = pl.ds(jj * L, L)
                    v = xrb0[r, sl] * rcv + biasv[sl]
                    xrb0[r, sl] = jnp.maximum(v, 0.0)
            pltpu.sync_copy(
                xrb0.at[pl.ds(0, rows)],
                outref.at[pl.ds(rowbase, rows), pl.ds(c * C, C)])

        def _wchunk(jw, _):
            k = tid + jw * NSUB

            @pl.when(k < WFULL)
            def _():
                _wb(CHUNK, k * CHUNK)

            @pl.when(k == WFULL)
            def _():
                _wb(WTAIL, WFULL * CHUNK)
            return 0
        lax.fori_loop(0, (WFULL + NSUB) // NSUB, _wchunk, 0)
        plsc.subcore_barrier()


_sc_call = pl.kernel(
    _sc_body,
    out_type=(
        jax.ShapeDtypeStruct((N, H * C), jnp.float32),
        jax.ShapeDtypeStruct((N, H * C), jnp.float32),
    ),
    mesh=plsc.VectorSubcoreMesh(core_axis_name="c", subcore_axis_name="s"),
    compiler_params=pltpu.CompilerParams(needs_layout_passes=False),
    scratch_types=[
        pltpu.VMEM((CHUNK, C), jnp.float32),
        pltpu.VMEM((CHUNK, C), jnp.float32),
        pltpu.VMEM((CHUNK, C), jnp.float32),
        pltpu.VMEM((CHUNK, C), jnp.float32),
        pltpu.VMEM((CHUNK, C), jnp.float32),
        pltpu.VMEM((2, CHUNK), jnp.int32),
        pltpu.VMEM((2, CHUNK), jnp.int32),
        pltpu.VMEM((2, CHUNK), jnp.int32),
        pltpu.VMEM((2, CHUNK), jnp.int32),
        pltpu.VMEM((CHUNK,), jnp.float32),
        pltpu.VMEM((CHUNK,), jnp.float32),
        pltpu.VMEM((C,), jnp.float32),
        pltpu.VMEM((C,), jnp.float32),
        pltpu.VMEM_SHARED((N, C), jnp.float32),
        pltpu.VMEM_SHARED((NPAD,), jnp.float32),
        pltpu.SemaphoreType.DMA,
        pltpu.SemaphoreType.DMA,
        pltpu.SemaphoreType.DMA,
        pltpu.SemaphoreType.DMA,
        pltpu.SemaphoreType.DMA,
        pltpu.SemaphoreType.DMA,
        pltpu.SemaphoreType.DMA,
        pltpu.SemaphoreType.DMA,
        pltpu.SemaphoreType.DMA,
        pltpu.SemaphoreType.DMA,
        pltpu.SemaphoreType.DMA,
        pltpu.SemaphoreType.DMA,
        pltpu.SemaphoreType.DMA,
    ],
)


@jax.jit
def kernel(s, t, edges, edge_weight,
           sWl, sWr, sWe, satt, sbias,
           tWl, tWr, tWe, tatt, tbias):
    pa = _project(s, jnp.concatenate([sWl, tWr], axis=1), 1000)
    pb = _project(t, jnp.concatenate([sWr, tWl], axis=1), 1000)
    paf = pa.reshape(4 * N, C)
    pbf = pb.reshape(4 * N, C)
    eef = _project_w(edge_weight, jnp.concatenate([sWe, tWe], axis=1),
                     8000)

    att4 = jnp.concatenate([satt, tatt], axis=0)
    attb = att4.reshape(4 * C)
    bias4 = jnp.concatenate(
        [sbias.reshape(H, C), tbias.reshape(H, C)], axis=0)
    biasb = bias4.reshape(4 * C)

    outs, outt = _sc_call(edges[0], edges[1], paf, pbf, eef, attb, biasb)
    return (outs, outt, edges, edge_weight)


# speedup vs baseline: 1.1253x; 1.0030x over previous
import functools

import jax
import jax.numpy as jnp
from jax import lax
from jax.experimental import pallas as pl
from jax.experimental.pallas import tpu as pltpu
from jax.experimental.pallas import tpu_sc as plsc

N = 10000
E = 160000
D = 256
H = 2
C = 128
NEG = 0.2
L = 16
NSUB = 16
CHUNK = 64
NGRP = CHUNK // L
NCHG = E // CHUNK
JPAD = (NCHG + NSUB - 1) // NSUB
NPAD = JPAD * CHUNK
WFULL = N // CHUNK
WTAIL = N - WFULL * CHUNK


def _mm_body(x_ref, w_ref, o_ref):
    o_ref[0] = jnp.dot(x_ref[...], w_ref[...],
                       preferred_element_type=jnp.float32)


def _mm_body2(x_ref, w_ref, o_ref):
    o_ref[...] = jnp.dot(x_ref[...], w_ref[...],
                         preferred_element_type=jnp.float32)


def _project_w(x, w, bn):
    M, K = x.shape
    G = w.shape[1] // 128
    nb = M // bn
    return pl.pallas_call(
        _mm_body2,
        grid=(G, nb),
        in_specs=[
            pl.BlockSpec((bn, K), lambda g, i: (i, 0)),
            pl.BlockSpec((K, 128), lambda g, i: (0, g)),
        ],
        out_specs=pl.BlockSpec((bn, 128), lambda g, i: (g * (M // bn) + i, 0)),
        out_shape=jax.ShapeDtypeStruct((G * M, 128), jnp.float32),
    )(x, w)


def _project(x, w, bn):
    M, K = x.shape
    G = w.shape[1] // 128
    return pl.pallas_call(
        _mm_body,
        grid=(G, M // bn),
        in_specs=[
            pl.BlockSpec((bn, K), lambda g, i: (i, 0)),
            pl.BlockSpec((K, 128), lambda g, i: (0, g)),
        ],
        out_specs=pl.BlockSpec((1, bn, 128), lambda g, i: (g, i, 0)),
        out_shape=jax.ShapeDtypeStruct((G, M, 128), jnp.float32),
    )(x, w)


def _sc_body(e0, e1, paf, pbf, eef, attb, biasb,
             outs, outt,
             xlb0, xlb1, xrb0, xrb1, eeb,
             eib0, eib1, eib2, eib3,
             exb0, exb1, attv, biasv,
             out_sh, denf_sh,
             semxl0, semxl1, semxr0, semxr1, semee,
             semso0, semso1, semsd0, semsd1,
             semei0, semei1, semei2, semei3):
    c = lax.axis_index("c")
    tid = lax.axis_index("s")
    iota = lax.iota(jnp.int32, L)
    zv = jnp.zeros((L,), jnp.float32)
    lane0 = iota == 0
    xlb = (xlb0, xlb1)
    xrb = (xrb0, xrb1)
    eib = (eib0, eib1, eib2, eib3)
    semei = (semei0, semei1, semei2, semei3)
    semxl = (semxl0, semxl1)
    semxr = (semxr0, semxr1)
    semso = (semso0, semso1)
    semsd = (semsd0, semsd1)
    exb = (exb0, exb1)

    for d, (xl, xr, esrc, edst, outref) in enumerate((
            (paf, pbf, e0, e1, outs),
            (pbf, paf, e1, e0, outt))):
        q = d * 2 + c
        pltpu.sync_copy(attb.at[pl.ds(q * C, C)], attv)
        pltpu.sync_copy(biasb.at[pl.ds(q * C, C)], biasv)
        hoff = q * N

        def _zrow(r, _):
            for jz in range(C // L):
                xlb0[r, pl.ds(jz * L, L)] = zv
            return 0
        lax.fori_loop(0, CHUNK, _zrow, 0)
        for g in range(NGRP):
            exb0[pl.ds(g * L, L)] = zv

        def _zout(jz, _):
            k = tid + jz * NSUB

            @pl.when(k < WFULL)
            def _():
                pltpu.sync_copy(xlb0, out_sh.at[pl.ds(k * CHUNK, CHUNK)])

            @pl.when(k == WFULL)
            def _():
                pltpu.sync_copy(xlb0.at[pl.ds(0, WTAIL)],
                                out_sh.at[pl.ds(WFULL * CHUNK, WTAIL)])

            @pl.when(k < JPAD)
            def _():
                pltpu.sync_copy(exb0, denf_sh.at[pl.ds(k * CHUNK, CHUNK)])
            return 0
        lax.fori_loop(0, (JPAD + NSUB - 1) // NSUB, _zout, 0)

        plsc.subcore_barrier()

        def _ifetch(j, qs):
            k = tid + j * NSUB
            base = jnp.minimum(k, NCHG - 1) * CHUNK
            pltpu.async_copy(esrc.at[pl.ds(base, CHUNK)], eib[qs].at[0],
                             semei[qs])
            pltpu.async_copy(edst.at[pl.ds(base, CHUNK)], eib[qs].at[1],
                             semei[qs])

        def _gissue(j, qs, b):
            k = tid + j * NSUB
            base = jnp.minimum(k, NCHG - 1) * CHUNK
            pltpu.make_async_copy(esrc.at[pl.ds(base, CHUNK)],
                                  eib[qs].at[0], semei[qs]).wait()
            pltpu.make_async_copy(edst.at[pl.ds(base, CHUNK)],
                                  eib[qs].at[1], semei[qs]).wait()
            for g in range(NGRP):
                sl = pl.ds(g * L, L)
                eib[qs][0, sl] = eib[qs][0, sl] + hoff
                eib[qs][1, sl] = eib[qs][1, sl] + hoff
            pltpu.async_copy(xl.at[eib[qs].at[0]], xlb[b], semxl[b])
            pltpu.async_copy(xr.at[eib[qs].at[1]], xrb[b], semxr[b])

        def _compute(j, qs, b):
            valid = (tid + j * NSUB) < NCHG
            vs = jnp.full((L,), jnp.where(valid, 1.0, 0.0), jnp.float32)
            mxl = xlb[b]
            mxr = xrb[b]
            mee = eeb

            @plsc.parallel_loop(0, CHUNK, unroll=4)
            def _alpha(e):
                acc = zv
                for jj in range(C // L):
                    sl = pl.ds(jj * L, L)
                    m = mxl[e, sl] + mxr[e, sl] + mee[e, sl]
                    m = jnp.maximum(m, NEG * m)
                    acc = acc + m * attv[sl]
                ex = jnp.exp(jnp.full((L,), jnp.sum(acc), jnp.float32)) * vs
                plsc.store_scatter(exb[b], [jnp.full((L,), e, jnp.int32)],
                                   ex, mask=lane0)

            @pl.when(j + 1 < JPAD)
            def _():
                k1 = tid + (j + 1) * NSUB
                base1 = jnp.minimum(k1, NCHG - 1) * CHUNK
                pltpu.async_copy(eef.at[pl.ds(q * E + base1, CHUNK)], eeb,
                                 semee)

            @plsc.parallel_loop(0, CHUNK, unroll=4)
            def _scale(e):
                exv = plsc.load_gather(exb[b],
                                       [jnp.full((L,), e, jnp.int32)])
                for jj in range(C // L):
                    sl = pl.ds(jj * L, L)
                    mxl[e, sl] = mxl[e, sl] * exv

            for g in range(NGRP):
                sl = pl.ds(g * L, L)
                eib[qs][1, sl] = eib[qs][1, sl] - hoff
            pltpu.async_copy(exb[b], denf_sh.at[eib[qs].at[1]], semsd[b],
                             add=True)
            pltpu.async_copy(mxl, out_sh.at[eib[qs].at[1]], semso[b],
                             add=True)

        _ifetch(0, 0)
        _ifetch(1, 1)
        _ifetch(2, 2)
        _gissue(0, 0, 0)
        base00 = jnp.minimum(tid, NCHG - 1) * CHUNK
        pltpu.async_copy(eef.at[pl.ds(q * E + base00, CHUNK)], eeb, semee)

        def _quad(jo, _):
            for b4 in range(4):
                j = jo * 4 + b4
                b = b4 % 2

                @pl.when(j < JPAD)
                def _():
                    k = tid + j * NSUB
                    base = jnp.minimum(k, NCHG - 1) * CHUNK
                    pltpu.make_async_copy(xl.at[eib[b4].at[0]], xlb[b],
                                          semxl[b]).wait()
                    pltpu.make_async_copy(xr.at[eib[b4].at[1]], xrb[b],
                                          semxr[b]).wait()
                    pltpu.make_async_copy(
                        eef.at[pl.ds(q * E + base, CHUNK)], eeb,
                        semee).wait()

                    @pl.when(j + 3 < JPAD)
                    def _():
                        _ifetch(j + 3, (b4 + 3) % 4)

                    @pl.when(j + 1 < JPAD)
                    def _():
                        @pl.when(j >= 1)
                        def _():
                            pltpu.make_async_copy(
                                exb[1 - b],
                                denf_sh.at[eib[(b4 + 3) % 4].at[1]],
                                semsd[1 - b]).wait()
                            pltpu.make_async_copy(
                                xlb[1 - b],
                                out_sh.at[eib[(b4 + 3) % 4].at[1]],
                                semso[1 - b]).wait()
                        _gissue(j + 1, (b4 + 1) % 4, 1 - b)
                    _compute(j, b4, b)
            return 0
        lax.fori_loop(0, (JPAD + 3) // 4, _quad, 0)
        for m in (JPAD - 2, JPAD - 1):
            bm = m % 2
            qm = m % 4
            pltpu.make_async_copy(exb[bm], denf_sh.at[eib[qm].at[1]],
                                  semsd[bm]).wait()
            pltpu.make_async_copy(xlb[bm], out_sh.at[eib[qm].at[1]],
                                  semso[bm]).wait()

        plsc.subcore_barrier()

        def _wb(rows, rowbase):
            pltpu.sync_copy(out_sh.at[pl.ds(rowbase, rows)],
                            xrb0.at[pl.ds(0, rows)])
            pltpu.sync_copy(denf_sh.at[pl.ds(rowbase, rows)],
                            exb0.at[pl.ds(0, rows)])

            @plsc.parallel_loop(0, rows, unroll=4)
            def _nrm(r):
                dv = plsc.load_gather(exb0,
                                      [jnp.full((L,), r, jnp.int32)])
                rcv = 1.0 / (dv + 1e-16)
                for jj in range(C // L):
                    sl = pl.ds(jj * L, L)
                    v = xrb0[r, sl] * rcv + biasv[sl]
                    xrb0[r, sl] = jnp.maximum(v, 0.0)
            pltpu.sync_copy(
                xrb0.at[pl.ds(0, rows)],
                outref.at[pl.ds(rowbase, rows), pl.ds(c * C, C)])

        def _wchunk(jw, _):
            k = tid + jw * NSUB

            @pl.when(k < WFULL)
            def _():
                _wb(CHUNK, k * CHUNK)

            @pl.when(k == WFULL)
            def _():
                _wb(WTAIL, WFULL * CHUNK)
            return 0
        lax.fori_loop(0, (WFULL + NSUB) // NSUB, _wchunk, 0)
        plsc.subcore_barrier()


_sc_call = pl.kernel(
    _sc_body,
    out_type=(
        jax.ShapeDtypeStruct((N, H * C), jnp.float32),
        jax.ShapeDtypeStruct((N, H * C), jnp.float32),
    ),
    mesh=plsc.VectorSubcoreMesh(core_axis_name="c", subcore_axis_name="s"),
    compiler_params=pltpu.CompilerParams(needs_layout_passes=False),
    scratch_types=[
        pltpu.VMEM((CHUNK, C), jnp.float32),
        pltpu.VMEM((CHUNK, C), jnp.float32),
        pltpu.VMEM((CHUNK, C), jnp.float32),
        pltpu.VMEM((CHUNK, C), jnp.float32),
        pltpu.VMEM((CHUNK, C), jnp.float32),
        pltpu.VMEM((2, CHUNK), jnp.int32),
        pltpu.VMEM((2, CHUNK), jnp.int32),
        pltpu.VMEM((2, CHUNK), jnp.int32),
        pltpu.VMEM((2, CHUNK), jnp.int32),
        pltpu.VMEM((CHUNK,), jnp.float32),
        pltpu.VMEM((CHUNK,), jnp.float32),
        pltpu.VMEM((C,), jnp.float32),
        pltpu.VMEM((C,), jnp.float32),
        pltpu.VMEM_SHARED((N, C), jnp.float32),
        pltpu.VMEM_SHARED((NPAD,), jnp.float32),
        pltpu.SemaphoreType.DMA,
        pltpu.SemaphoreType.DMA,
        pltpu.SemaphoreType.DMA,
        pltpu.SemaphoreType.DMA,
        pltpu.SemaphoreType.DMA,
        pltpu.SemaphoreType.DMA,
        pltpu.SemaphoreType.DMA,
        pltpu.SemaphoreType.DMA,
        pltpu.SemaphoreType.DMA,
        pltpu.SemaphoreType.DMA,
        pltpu.SemaphoreType.DMA,
        pltpu.SemaphoreType.DMA,
        pltpu.SemaphoreType.DMA,
    ],
)


@jax.jit
def kernel(s, t, edges, edge_weight,
           sWl, sWr, sWe, satt, sbias,
           tWl, tWr, tWe, tatt, tbias):
    pa = _project(s, jnp.concatenate([sWl, tWr], axis=1), 1000)
    pb = _project(t, jnp.concatenate([sWr, tWl], axis=1), 1000)
    paf = pa.reshape(4 * N, C)
    pbf = pb.reshape(4 * N, C)
    eef = _project_w(edge_weight, jnp.concatenate([sWe, tWe], axis=1),
                     16000)

    att4 = jnp.concatenate([satt, tatt], axis=0)
    attb = att4.reshape(4 * C)
    bias4 = jnp.concatenate(
        [sbias.reshape(H, C), tbias.reshape(H, C)], axis=0)
    biasb = bias4.reshape(4 * C)

    outs, outt = _sc_call(edges[0], edges[1], paf, pbf, eef, attb, biasb)
    return (outs, outt, edges, edge_weight)
